# Initial kernel scaffold; baseline (speedup 1.0000x reference)
import jax
import jax.numpy as jnp
from jax.experimental import pallas as pl


def kernel(x, c, layer0_wmat, layer0_bias, layer1_wmat, layer1_bias, layer1_gamma, layer1_beta, layer2_wmat, layer2_bias, layer2_gamma, layer2_beta, layer3_wmat, layer3_bias, layer3_gamma, layer3_beta, layer4_wmat, layer4_bias, layer4_gamma, layer4_beta, head_fout_w, head_fout_b, head_embed_w, head_embed_b):
    raise NotImplementedError("write your pallas kernel here")


# speedup vs baseline: 8.9377x; 8.9377x over previous
import jax
import jax.numpy as jnp
from jax.experimental import pallas as pl
from jax.experimental.pallas import tpu as pltpu

BN_EPS = 1e-5
_VMEM_LIMIT = 48 * 1024 * 1024


def _l0_kernel(p_ref, w_ref, b_ref, o_ref, *, H, W, TM):
    half = o_ref.shape[-1]
    dt = o_ref.dtype
    o_ref[0, 0:1, :, :] = jnp.zeros((1, W + 2, half), dt)
    o_ref[0, H + 1 : H + 2, :, :] = jnp.zeros((1, W + 2, half), dt)
    o_ref[0, :, 0:1, :] = jnp.zeros((H + 2, 1, half), dt)
    o_ref[0, :, W + 1 : W + 2, :] = jnp.zeros((H + 2, 1, half), dt)
    rows = TM // W
    for k in range(p_ref.shape[1] // TM):
        z = jnp.dot(
            p_ref[0, k * TM : (k + 1) * TM, :],
            w_ref[...],
            preferred_element_type=jnp.float32,
        )
        z = z + b_ref[...]
        o = (z[:, :half] * jax.nn.sigmoid(z[:, half:])).astype(o_ref.dtype)
        o3 = o.reshape(rows, W, half)
        r0 = 1 + k * rows
        o_ref[0, r0 : r0 + rows, 1 : W + 1, :] = o3


def _conv_s2_pass1_kernel(x_ref, w_ref, y_ref, s1_ref, s2_ref, a_ref, *, Ho, Wo, C, TOH):
    TM = TOH * Wo
    F2 = w_ref.shape[1]
    s1 = jnp.zeros((1, F2), jnp.float32)
    s2 = jnp.zeros((1, F2), jnp.float32)
    ones = jnp.ones((1, TM), jnp.float32)
    for k in range(Ho // TOH):
        buf = k % 2
        oh0 = k * TOH
        for i in range(3):
            di, pi = i // 2, i % 2
            for j in range(3):
                t = i * 3 + j
                dj, q = j // 2, j % 2
                sl = x_ref[0, oh0 + di : oh0 + di + TOH, pi,
                           dj : dj + Wo, q * C : (q + 1) * C]
                a_ref[buf, :, t * C : (t + 1) * C] = sl.reshape(TM, C)
        z = jnp.dot(a_ref[buf], w_ref[...], preferred_element_type=jnp.float32)
        s1 = s1 + jnp.dot(ones, z, preferred_element_type=jnp.float32)
        s2 = s2 + jnp.dot(ones, z * z, preferred_element_type=jnp.float32)
        y_ref[0, k * TM : (k + 1) * TM, :] = z.astype(y_ref.dtype)
    s1_ref[0] = s1
    s2_ref[0] = s2


def _conv5x1_pass1_kernel(x_ref, w_ref, y_ref, s1_ref, s2_ref, a_ref, *, Ho, Wo, C, TOH):
    TM = TOH * Wo
    F2 = w_ref.shape[1]
    s1 = jnp.zeros((1, F2), jnp.float32)
    s2 = jnp.zeros((1, F2), jnp.float32)
    ones = jnp.ones((1, TM), jnp.float32)
    for k in range(Ho // TOH):
        buf = k % 2
        oh0 = k * TOH
        for i in range(5):
            sl = x_ref[0, oh0 + i : oh0 + i + TOH, :, :]
            a_ref[buf, :, i * C : (i + 1) * C] = sl.reshape(TM, C)
        z = jnp.dot(a_ref[buf], w_ref[...], preferred_element_type=jnp.float32)
        s1 = s1 + jnp.dot(ones, z, preferred_element_type=jnp.float32)
        s2 = s2 + jnp.dot(ones, z * z, preferred_element_type=jnp.float32)
        y_ref[0, k * TM : (k + 1) * TM, :] = z.astype(y_ref.dtype)
    s1_ref[0] = s1
    s2_ref[0] = s2


def _bn_glu_pad_kernel(y_ref, sc_ref, sh_ref, o_ref, *, Ho, Wo, TOH, ph, pw):
    half = o_ref.shape[-1]
    Hp = o_ref.shape[1]
    Wp = o_ref.shape[2]
    dt = o_ref.dtype
    if ph:
        o_ref[0, 0:ph, :, :] = jnp.zeros((ph, Wp, half), dt)
        o_ref[0, ph + Ho : Hp, :, :] = jnp.zeros((Hp - ph - Ho, Wp, half), dt)
    if pw:
        o_ref[0, :, 0:pw, :] = jnp.zeros((Hp, pw, half), dt)
        o_ref[0, :, pw + Wo : Wp, :] = jnp.zeros((Hp, Wp - pw - Wo, half), dt)
    TM = TOH * Wo
    sc = sc_ref[...]
    sh = sh_ref[...]
    for k in range(Ho // TOH):
        yv = y_ref[0, k * TM : (k + 1) * TM, :].astype(jnp.float32)
        yn = yv * sc + sh
        o = (yn[:, :half] * jax.nn.sigmoid(yn[:, half:])).astype(o_ref.dtype)
        o3 = o.reshape(TOH, Wo, half)
        r0 = ph + k * TOH
        o_ref[0, r0 : r0 + TOH, pw : pw + Wo, :] = o3


def _head_kernel(y_ref, sc_ref, sh_ref, c_ref, fw_ref, fb_ref, ew_ref, eb_ref, o_ref):
    N, M, F2 = y_ref.shape
    half = F2 // 2
    sc = sc_ref[...]
    sh = sh_ref[...]
    chunk = min(128, M)
    ones = jnp.ones((1, chunk), jnp.float32)
    hs_rows = []
    for n in range(N):
        acc = jnp.zeros((1, half), jnp.float32)
        for k in range(M // chunk):
            yv = y_ref[n, k * chunk : (k + 1) * chunk, :].astype(jnp.float32)
            yn = yv * sc + sh
            h = yn[:, :half] * jax.nn.sigmoid(yn[:, half:])
            acc = acc + jnp.dot(ones, h, preferred_element_type=jnp.float32)
        hs_rows.append(acc)
    hs = jnp.concatenate(hs_rows, axis=0)
    e = jnp.dot(c_ref[...], ew_ref[...], preferred_element_type=jnp.float32) + eb_ref[...]
    t = hs * (e + fw_ref[...])
    o_ref[...] = jnp.sum(t, axis=1, keepdims=True) + fb_ref[...]


def _conv_s2_layer(xp, wmat, gamma, beta, Ho, Wo):
    N, Hp, Wp, C = xp.shape
    K, F2 = wmat.shape
    half = F2 // 2
    M_img = Ho * Wo
    TOH = min(4, Ho)
    TM = TOH * Wo
    x5 = xp.reshape(N, Hp // 2, 2, Wp // 2, 2 * C)

    y, s1p, s2p = pl.pallas_call(
        lambda x_ref, w_ref, y_ref, s1_ref, s2_ref, a_ref: _conv_s2_pass1_kernel(
            x_ref, w_ref, y_ref, s1_ref, s2_ref, a_ref, Ho=Ho, Wo=Wo, C=C, TOH=TOH
        ),
        out_shape=(
            jax.ShapeDtypeStruct((N, M_img, F2), jnp.bfloat16),
            jax.ShapeDtypeStruct((N, 1, F2), jnp.float32),
            jax.ShapeDtypeStruct((N, 1, F2), jnp.float32),
        ),
        grid=(N,),
        in_specs=[
            pl.BlockSpec((1, Hp // 2, 2, Wp // 2, 2 * C), lambda n: (n, 0, 0, 0, 0)),
            pl.BlockSpec((K, F2), lambda n: (0, 0)),
        ],
        out_specs=(
            pl.BlockSpec((1, M_img, F2), lambda n: (n, 0, 0)),
            pl.BlockSpec((1, 1, F2), lambda n: (n, 0, 0)),
            pl.BlockSpec((1, 1, F2), lambda n: (n, 0, 0)),
        ),
        scratch_shapes=[pltpu.VMEM((2, TM, K), jnp.bfloat16)],
        compiler_params=pltpu.CompilerParams(
            dimension_semantics=("parallel",), vmem_limit_bytes=_VMEM_LIMIT
        ),
        cost_estimate=pl.CostEstimate(
            flops=2 * N * M_img * K * F2,
            transcendentals=0,
            bytes_accessed=xp.size * 2 + wmat.size * 2 + N * M_img * F2 * 2,
        ),
    )(x5, wmat)

    M_tot = N * M_img
    mean = jnp.sum(s1p, axis=0) / M_tot
    var = jnp.maximum(jnp.sum(s2p, axis=0) / M_tot - mean * mean, 0.0)
    scale = gamma * jax.lax.rsqrt(var + BN_EPS)
    shift = beta - mean * scale
    return y, scale, shift, half


def _bn_glu_pad(y, scale, shift, Ho, Wo, half, ph, pw):
    N, M_img, F2 = y.shape
    TOH = min(4, Ho)
    Hp, Wp = Ho + 2 * ph, Wo + 2 * pw
    return pl.pallas_call(
        lambda y_ref, sc_ref, sh_ref, o_ref: _bn_glu_pad_kernel(
            y_ref, sc_ref, sh_ref, o_ref, Ho=Ho, Wo=Wo, TOH=TOH, ph=ph, pw=pw
        ),
        out_shape=jax.ShapeDtypeStruct((N, Hp, Wp, half), jnp.bfloat16),
        grid=(N,),
        in_specs=[
            pl.BlockSpec((1, M_img, F2), lambda n: (n, 0, 0)),
            pl.BlockSpec((1, F2), lambda n: (0, 0)),
            pl.BlockSpec((1, F2), lambda n: (0, 0)),
        ],
        out_specs=pl.BlockSpec((1, Hp, Wp, half), lambda n: (n, 0, 0, 0)),
        compiler_params=pltpu.CompilerParams(
            dimension_semantics=("parallel",), vmem_limit_bytes=_VMEM_LIMIT
        ),
        cost_estimate=pl.CostEstimate(
            flops=3 * N * M_img * F2,
            transcendentals=N * M_img * half,
            bytes_accessed=y.size * 2 + N * Hp * Wp * half * 2,
        ),
    )(y, scale, shift)


def kernel(x, c,
           layer0_wmat, layer0_bias,
           layer1_wmat, layer1_bias, layer1_gamma, layer1_beta,
           layer2_wmat, layer2_bias, layer2_gamma, layer2_beta,
           layer3_wmat, layer3_bias, layer3_gamma, layer3_beta,
           layer4_wmat, layer4_bias, layer4_gamma, layer4_beta,
           head_fout_w, head_fout_b, head_embed_w, head_embed_b):
    N, _, H, W = x.shape

    xb = jnp.pad(x.reshape(N, H, W).astype(jnp.bfloat16), ((0, 0), (1, 1), (1, 1)))
    taps = [xb[:, i : i + H, j : j + W] for i in range(3) for j in range(3)]
    patches = jnp.stack(taps, axis=-1).reshape(N, H * W, 9)
    F0 = layer0_wmat.shape[1]
    half0 = F0 // 2
    TM0 = min(1024, H * W)
    h0 = pl.pallas_call(
        lambda p_ref, w_ref, b_ref, o_ref: _l0_kernel(
            p_ref, w_ref, b_ref, o_ref, H=H, W=W, TM=TM0
        ),
        out_shape=jax.ShapeDtypeStruct((N, H + 2, W + 2, half0), jnp.bfloat16),
        grid=(N,),
        in_specs=[
            pl.BlockSpec((1, H * W, 9), lambda n: (n, 0, 0)),
            pl.BlockSpec((9, F0), lambda n: (0, 0)),
            pl.BlockSpec((1, F0), lambda n: (0, 0)),
        ],
        out_specs=pl.BlockSpec((1, H + 2, W + 2, half0), lambda n: (n, 0, 0, 0)),
        compiler_params=pltpu.CompilerParams(
            dimension_semantics=("parallel",), vmem_limit_bytes=_VMEM_LIMIT
        ),
        cost_estimate=pl.CostEstimate(
            flops=2 * N * H * W * 9 * F0,
            transcendentals=N * H * W * half0,
            bytes_accessed=patches.size * 2 + N * (H + 2) * (W + 2) * half0 * 2,
        ),
    )(patches, layer0_wmat, layer0_bias)

    xp = h0
    Ho, Wo = H, W
    for wmat, gamma, beta in (
        (layer1_wmat, layer1_gamma, layer1_beta),
        (layer2_wmat, layer2_gamma, layer2_beta),
        (layer3_wmat, layer3_gamma, layer3_beta),
    ):
        Ho, Wo = Ho // 2, Wo // 2
        y, scale, shift, half = _conv_s2_layer(xp, wmat, gamma, beta, Ho, Wo)
        last = wmat is layer3_wmat
        ph, pw = (2, 0) if last else (1, 1)
        xp = _bn_glu_pad(y, scale, shift, Ho, Wo, half, ph, pw)

    N4, Hp4, Wp4, C4 = xp.shape
    K4, F4 = layer4_wmat.shape
    M4 = Ho * Wo
    TOH4 = min(4, Ho)
    y4, s1p, s2p = pl.pallas_call(
        lambda x_ref, w_ref, y_ref, s1_ref, s2_ref, a_ref: _conv5x1_pass1_kernel(
            x_ref, w_ref, y_ref, s1_ref, s2_ref, a_ref, Ho=Ho, Wo=Wo, C=C4, TOH=TOH4
        ),
        out_shape=(
            jax.ShapeDtypeStruct((N, M4, F4), jnp.bfloat16),
            jax.ShapeDtypeStruct((N, 1, F4), jnp.float32),
            jax.ShapeDtypeStruct((N, 1, F4), jnp.float32),
        ),
        grid=(N,),
        in_specs=[
            pl.BlockSpec((1, Hp4, Wp4, C4), lambda n: (n, 0, 0, 0)),
            pl.BlockSpec((K4, F4), lambda n: (0, 0)),
        ],
        out_specs=(
            pl.BlockSpec((1, M4, F4), lambda n: (n, 0, 0)),
            pl.BlockSpec((1, 1, F4), lambda n: (n, 0, 0)),
            pl.BlockSpec((1, 1, F4), lambda n: (n, 0, 0)),
        ),
        scratch_shapes=[pltpu.VMEM((2, TOH4 * Wo, K4), jnp.bfloat16)],
        compiler_params=pltpu.CompilerParams(
            dimension_semantics=("parallel",), vmem_limit_bytes=_VMEM_LIMIT
        ),
        cost_estimate=pl.CostEstimate(
            flops=2 * N * M4 * K4 * F4,
            transcendentals=0,
            bytes_accessed=xp.size * 2 + layer4_wmat.size * 2 + N * M4 * F4 * 2,
        ),
    )(xp, layer4_wmat)

    M_tot = N * M4
    mean = jnp.sum(s1p, axis=0) / M_tot
    var = jnp.maximum(jnp.sum(s2p, axis=0) / M_tot - mean * mean, 0.0)
    scale4 = layer4_gamma * jax.lax.rsqrt(var + BN_EPS)
    shift4 = layer4_beta - mean * scale4

    fwT = head_fout_w.reshape(1, -1)
    out = pl.pallas_call(
        _head_kernel,
        out_shape=jax.ShapeDtypeStruct((N, 1), jnp.float32),
        in_specs=[pl.BlockSpec(memory_space=pltpu.MemorySpace.VMEM)] * 8,
        out_specs=pl.BlockSpec(memory_space=pltpu.MemorySpace.VMEM),
        compiler_params=pltpu.CompilerParams(vmem_limit_bytes=_VMEM_LIMIT),
        cost_estimate=pl.CostEstimate(
            flops=6 * N * M4 * F4,
            transcendentals=N * M4 * F4 // 2,
            bytes_accessed=y4.size * 2,
        ),
    )(y4, scale4, shift4, c.astype(jnp.float32), fwT, head_fout_b,
      head_embed_w, head_embed_b)
    return out


# speedup vs baseline: 9.0330x; 1.0107x over previous
import jax
import jax.numpy as jnp
from jax.experimental import pallas as pl
from jax.experimental.pallas import tpu as pltpu

BN_EPS = 1e-5
_VMEM_LIMIT = 48 * 1024 * 1024


def _l0_kernel(p_ref, w_ref, b_ref, o_ref, *, H, W, TM):
    half = o_ref.shape[-1]
    dt = o_ref.dtype
    o_ref[0, 0:1, :, :] = jnp.zeros((1, W + 2, half), dt)
    o_ref[0, H + 1 : H + 2, :, :] = jnp.zeros((1, W + 2, half), dt)
    o_ref[0, :, 0:1, :] = jnp.zeros((H + 2, 1, half), dt)
    o_ref[0, :, W + 1 : W + 2, :] = jnp.zeros((H + 2, 1, half), dt)
    rows = TM // W
    for k in range(p_ref.shape[1] // TM):
        z = jnp.dot(
            p_ref[0, k * TM : (k + 1) * TM, :],
            w_ref[...],
            preferred_element_type=jnp.float32,
        )
        z = z + b_ref[...]
        o = (z[:, :half] * jax.nn.sigmoid(z[:, half:])).astype(o_ref.dtype)
        o3 = o.reshape(rows, W, half)
        r0 = 1 + k * rows
        o_ref[0, r0 : r0 + rows, 1 : W + 1, :] = o3


def _tap_slice(x_ref, im, oh0, TOH, Wo, C, tap, paired):
    if paired:
        di, pi, dj, q = tap
        sl = x_ref[im, oh0 + di : oh0 + di + TOH, pi,
                   dj : dj + Wo, q * C : (q + 1) * C]
    else:
        (i,) = tap
        sl = x_ref[im, oh0 + i : oh0 + i + TOH, :, :]
    return sl.reshape(TOH * Wo, C)


def _conv_pass1_kernel(x_ref, w_ref, y_ref, s1_ref, s2_ref, a_ref, *,
                       taps, Ho, Wo, C, TOH, IMGS, G, paired):
    M_img = Ho * Wo
    TMi = TOH * Wo
    R = G * TMi
    F2 = w_ref.shape[1]
    NT = min(256, F2)
    pieces = [(im, oh0) for im in range(IMGS) for oh0 in range(0, Ho, TOH)]
    n_chunks = len(pieces) // G
    nbuf = a_ref.shape[0]
    ones = jnp.ones((1, R), jnp.float32)
    s1 = [jnp.zeros((1, NT), jnp.float32) for _ in range(F2 // NT)]
    s2 = [jnp.zeros((1, NT), jnp.float32) for _ in range(F2 // NT)]
    for g in range(n_chunks):
        buf = g % nbuf
        grp = pieces[g * G : (g + 1) * G]
        for gi, (im, oh0) in enumerate(grp):
            for t, tap in enumerate(taps):
                a_ref[buf, gi * TMi : (gi + 1) * TMi, t * C : (t + 1) * C] = (
                    _tap_slice(x_ref, im, oh0, TOH, Wo, C, tap, paired)
                )
        for nt in range(F2 // NT):
            z = jnp.dot(a_ref[buf], w_ref[:, nt * NT : (nt + 1) * NT],
                        preferred_element_type=jnp.float32)
            s1[nt] = s1[nt] + jnp.dot(ones, z, preferred_element_type=jnp.float32)
            s2[nt] = s2[nt] + jnp.dot(ones, z * z, preferred_element_type=jnp.float32)
            zb = z.astype(y_ref.dtype)
            for gi, (im, oh0) in enumerate(grp):
                y_ref[im, oh0 * Wo : oh0 * Wo + TMi, nt * NT : (nt + 1) * NT] = (
                    zb[gi * TMi : (gi + 1) * TMi, :]
                )
    for nt in range(F2 // NT):
        s1_ref[0, 0:1, nt * NT : (nt + 1) * NT] = s1[nt]
        s2_ref[0, 0:1, nt * NT : (nt + 1) * NT] = s2[nt]


_S2_TAPS = tuple((i // 2, i % 2, j // 2, j % 2) for i in range(3) for j in range(3))
_51_TAPS = tuple((i,) for i in range(5))


def _pass1_geometry(N, M_img, Ho, Wo):
    if M_img >= 512:
        IMGS, G, TOH = 1, 1, max(1, 512 // Wo)
    else:
        IMGS = max(1, min(N, 512 // M_img))
        G, TOH = IMGS, Ho
    return IMGS, G, TOH


def _bn_glu_pad_kernel(y_ref, sc_ref, sh_ref, o_ref, *, Ho, Wo, TOH, ph, pw):
    half = o_ref.shape[-1]
    Hp = o_ref.shape[1]
    Wp = o_ref.shape[2]
    dt = o_ref.dtype
    if ph:
        o_ref[0, 0:ph, :, :] = jnp.zeros((ph, Wp, half), dt)
        o_ref[0, ph + Ho : Hp, :, :] = jnp.zeros((Hp - ph - Ho, Wp, half), dt)
    if pw:
        o_ref[0, :, 0:pw, :] = jnp.zeros((Hp, pw, half), dt)
        o_ref[0, :, pw + Wo : Wp, :] = jnp.zeros((Hp, Wp - pw - Wo, half), dt)
    TM = TOH * Wo
    sc = sc_ref[...]
    sh = sh_ref[...]
    for k in range(Ho // TOH):
        yv = y_ref[0, k * TM : (k + 1) * TM, :].astype(jnp.float32)
        yn = yv * sc + sh
        o = (yn[:, :half] * jax.nn.sigmoid(yn[:, half:])).astype(o_ref.dtype)
        o3 = o.reshape(TOH, Wo, half)
        r0 = ph + k * TOH
        o_ref[0, r0 : r0 + TOH, pw : pw + Wo, :] = o3


def _head_kernel(y_ref, sc_ref, sh_ref, c_ref, fw_ref, fb_ref, ew_ref, eb_ref, o_ref):
    N, M, F2 = y_ref.shape
    half = F2 // 2
    sc = sc_ref[...]
    sh = sh_ref[...]
    chunk = min(128, M)
    ones = jnp.ones((1, chunk), jnp.float32)
    hs_rows = []
    for n in range(N):
        acc = jnp.zeros((1, half), jnp.float32)
        for k in range(M // chunk):
            yv = y_ref[n, k * chunk : (k + 1) * chunk, :].astype(jnp.float32)
            yn = yv * sc + sh
            h = yn[:, :half] * jax.nn.sigmoid(yn[:, half:])
            acc = acc + jnp.dot(ones, h, preferred_element_type=jnp.float32)
        hs_rows.append(acc)
    hs = jnp.concatenate(hs_rows, axis=0)
    e = jnp.dot(c_ref[...], ew_ref[...], preferred_element_type=jnp.float32) + eb_ref[...]
    t = hs * (e + fw_ref[...])
    o_ref[...] = jnp.sum(t, axis=1, keepdims=True) + fb_ref[...]


def _conv_s2_layer(xp, wmat, gamma, beta, Ho, Wo):
    N, Hp, Wp, C = xp.shape
    K, F2 = wmat.shape
    half = F2 // 2
    M_img = Ho * Wo
    IMGS, G, TOH = _pass1_geometry(N, M_img, Ho, Wo)
    R = G * TOH * Wo
    n_chunks = IMGS * M_img // R
    steps = N // IMGS
    x5 = xp.reshape(N, Hp // 2, 2, Wp // 2, 2 * C)

    y, s1p, s2p = pl.pallas_call(
        lambda x_ref, w_ref, y_ref, s1_ref, s2_ref, a_ref: _conv_pass1_kernel(
            x_ref, w_ref, y_ref, s1_ref, s2_ref, a_ref,
            taps=_S2_TAPS, Ho=Ho, Wo=Wo, C=C, TOH=TOH, IMGS=IMGS, G=G, paired=True,
        ),
        out_shape=(
            jax.ShapeDtypeStruct((N, M_img, F2), jnp.bfloat16),
            jax.ShapeDtypeStruct((steps, 1, F2), jnp.float32),
            jax.ShapeDtypeStruct((steps, 1, F2), jnp.float32),
        ),
        grid=(steps,),
        in_specs=[
            pl.BlockSpec((IMGS, Hp // 2, 2, Wp // 2, 2 * C),
                         lambda s: (s, 0, 0, 0, 0)),
            pl.BlockSpec((K, F2), lambda s: (0, 0)),
        ],
        out_specs=(
            pl.BlockSpec((IMGS, M_img, F2), lambda s: (s, 0, 0)),
            pl.BlockSpec((1, 1, F2), lambda s: (s, 0, 0)),
            pl.BlockSpec((1, 1, F2), lambda s: (s, 0, 0)),
        ),
        scratch_shapes=[pltpu.VMEM((2 if n_chunks > 1 else 1, R, K), jnp.bfloat16)],
        compiler_params=pltpu.CompilerParams(
            dimension_semantics=("parallel",), vmem_limit_bytes=_VMEM_LIMIT
        ),
        cost_estimate=pl.CostEstimate(
            flops=2 * N * M_img * K * F2,
            transcendentals=0,
            bytes_accessed=xp.size * 2 + wmat.size * 2 + N * M_img * F2 * 2,
        ),
    )(x5, wmat)

    M_tot = N * M_img
    mean = jnp.sum(s1p, axis=0) / M_tot
    var = jnp.maximum(jnp.sum(s2p, axis=0) / M_tot - mean * mean, 0.0)
    scale = gamma * jax.lax.rsqrt(var + BN_EPS)
    shift = beta - mean * scale
    return y, scale, shift, half


def _bn_glu_pad(y, scale, shift, Ho, Wo, half, ph, pw):
    N, M_img, F2 = y.shape
    TOH = min(4, Ho)
    Hp, Wp = Ho + 2 * ph, Wo + 2 * pw
    return pl.pallas_call(
        lambda y_ref, sc_ref, sh_ref, o_ref: _bn_glu_pad_kernel(
            y_ref, sc_ref, sh_ref, o_ref, Ho=Ho, Wo=Wo, TOH=TOH, ph=ph, pw=pw
        ),
        out_shape=jax.ShapeDtypeStruct((N, Hp, Wp, half), jnp.bfloat16),
        grid=(N,),
        in_specs=[
            pl.BlockSpec((1, M_img, F2), lambda n: (n, 0, 0)),
            pl.BlockSpec((1, F2), lambda n: (0, 0)),
            pl.BlockSpec((1, F2), lambda n: (0, 0)),
        ],
        out_specs=pl.BlockSpec((1, Hp, Wp, half), lambda n: (n, 0, 0, 0)),
        compiler_params=pltpu.CompilerParams(
            dimension_semantics=("parallel",), vmem_limit_bytes=_VMEM_LIMIT
        ),
        cost_estimate=pl.CostEstimate(
            flops=3 * N * M_img * F2,
            transcendentals=N * M_img * half,
            bytes_accessed=y.size * 2 + N * Hp * Wp * half * 2,
        ),
    )(y, scale, shift)


def kernel(x, c,
           layer0_wmat, layer0_bias,
           layer1_wmat, layer1_bias, layer1_gamma, layer1_beta,
           layer2_wmat, layer2_bias, layer2_gamma, layer2_beta,
           layer3_wmat, layer3_bias, layer3_gamma, layer3_beta,
           layer4_wmat, layer4_bias, layer4_gamma, layer4_beta,
           head_fout_w, head_fout_b, head_embed_w, head_embed_b):
    N, _, H, W = x.shape

    xb = jnp.pad(x.reshape(N, H, W).astype(jnp.bfloat16), ((0, 0), (1, 1), (1, 1)))
    taps = [xb[:, i : i + H, j : j + W] for i in range(3) for j in range(3)]
    patches = jnp.stack(taps, axis=-1).reshape(N, H * W, 9)
    F0 = layer0_wmat.shape[1]
    half0 = F0 // 2
    TM0 = min(1024, H * W)
    h0 = pl.pallas_call(
        lambda p_ref, w_ref, b_ref, o_ref: _l0_kernel(
            p_ref, w_ref, b_ref, o_ref, H=H, W=W, TM=TM0
        ),
        out_shape=jax.ShapeDtypeStruct((N, H + 2, W + 2, half0), jnp.bfloat16),
        grid=(N,),
        in_specs=[
            pl.BlockSpec((1, H * W, 9), lambda n: (n, 0, 0)),
            pl.BlockSpec((9, F0), lambda n: (0, 0)),
            pl.BlockSpec((1, F0), lambda n: (0, 0)),
        ],
        out_specs=pl.BlockSpec((1, H + 2, W + 2, half0), lambda n: (n, 0, 0, 0)),
        compiler_params=pltpu.CompilerParams(
            dimension_semantics=("parallel",), vmem_limit_bytes=_VMEM_LIMIT
        ),
        cost_estimate=pl.CostEstimate(
            flops=2 * N * H * W * 9 * F0,
            transcendentals=N * H * W * half0,
            bytes_accessed=patches.size * 2 + N * (H + 2) * (W + 2) * half0 * 2,
        ),
    )(patches, layer0_wmat, layer0_bias)

    xp = h0
    Ho, Wo = H, W
    for wmat, gamma, beta in (
        (layer1_wmat, layer1_gamma, layer1_beta),
        (layer2_wmat, layer2_gamma, layer2_beta),
        (layer3_wmat, layer3_gamma, layer3_beta),
    ):
        Ho, Wo = Ho // 2, Wo // 2
        y, scale, shift, half = _conv_s2_layer(xp, wmat, gamma, beta, Ho, Wo)
        last = wmat is layer3_wmat
        ph, pw = (2, 0) if last else (1, 1)
        xp = _bn_glu_pad(y, scale, shift, Ho, Wo, half, ph, pw)

    N4, Hp4, Wp4, C4 = xp.shape
    K4, F4 = layer4_wmat.shape
    M4 = Ho * Wo
    IMGS4, G4, TOH4 = _pass1_geometry(N, M4, Ho, Wo)
    R4 = G4 * TOH4 * Wo
    n_chunks4 = IMGS4 * M4 // R4
    steps4 = N // IMGS4
    y4, s1p, s2p = pl.pallas_call(
        lambda x_ref, w_ref, y_ref, s1_ref, s2_ref, a_ref: _conv_pass1_kernel(
            x_ref, w_ref, y_ref, s1_ref, s2_ref, a_ref,
            taps=_51_TAPS, Ho=Ho, Wo=Wo, C=C4, TOH=TOH4, IMGS=IMGS4, G=G4,
            paired=False,
        ),
        out_shape=(
            jax.ShapeDtypeStruct((N, M4, F4), jnp.bfloat16),
            jax.ShapeDtypeStruct((steps4, 1, F4), jnp.float32),
            jax.ShapeDtypeStruct((steps4, 1, F4), jnp.float32),
        ),
        grid=(steps4,),
        in_specs=[
            pl.BlockSpec((IMGS4, Hp4, Wp4, C4), lambda s: (s, 0, 0, 0)),
            pl.BlockSpec((K4, F4), lambda s: (0, 0)),
        ],
        out_specs=(
            pl.BlockSpec((IMGS4, M4, F4), lambda s: (s, 0, 0)),
            pl.BlockSpec((1, 1, F4), lambda s: (s, 0, 0)),
            pl.BlockSpec((1, 1, F4), lambda s: (s, 0, 0)),
        ),
        scratch_shapes=[
            pltpu.VMEM((2 if n_chunks4 > 1 else 1, R4, K4), jnp.bfloat16)
        ],
        compiler_params=pltpu.CompilerParams(
            dimension_semantics=("parallel",), vmem_limit_bytes=_VMEM_LIMIT
        ),
        cost_estimate=pl.CostEstimate(
            flops=2 * N * M4 * K4 * F4,
            transcendentals=0,
            bytes_accessed=xp.size * 2 + layer4_wmat.size * 2 + N * M4 * F4 * 2,
        ),
    )(xp, layer4_wmat)

    M_tot = N * M4
    mean = jnp.sum(s1p, axis=0) / M_tot
    var = jnp.maximum(jnp.sum(s2p, axis=0) / M_tot - mean * mean, 0.0)
    scale4 = layer4_gamma * jax.lax.rsqrt(var + BN_EPS)
    shift4 = layer4_beta - mean * scale4

    fwT = head_fout_w.reshape(1, -1)
    out = pl.pallas_call(
        _head_kernel,
        out_shape=jax.ShapeDtypeStruct((N, 1), jnp.float32),
        in_specs=[pl.BlockSpec(memory_space=pltpu.MemorySpace.VMEM)] * 8,
        out_specs=pl.BlockSpec(memory_space=pltpu.MemorySpace.VMEM),
        compiler_params=pltpu.CompilerParams(vmem_limit_bytes=_VMEM_LIMIT),
        cost_estimate=pl.CostEstimate(
            flops=6 * N * M4 * F4,
            transcendentals=N * M4 * F4 // 2,
            bytes_accessed=y4.size * 2,
        ),
    )(y4, scale4, shift4, c.astype(jnp.float32), fwT, head_fout_b,
      head_embed_w, head_embed_b)
    return out


# speedup vs baseline: 12.8651x; 1.4242x over previous
import jax
import jax.numpy as jnp
from jax.experimental import pallas as pl
from jax.experimental.pallas import tpu as pltpu

BN_EPS = 1e-5
_VMEM_LIMIT = 48 * 1024 * 1024


def _l0_kernel(p_ref, w_ref, b_ref, o_ref, *, H, W, TM):
    half = o_ref.shape[-1]
    dt = o_ref.dtype
    o_ref[0, 0:1, :, :] = jnp.zeros((1, W + 2, half), dt)
    o_ref[0, H + 1 : H + 2, :, :] = jnp.zeros((1, W + 2, half), dt)
    o_ref[0, :, 0:1, :] = jnp.zeros((H + 2, 1, half), dt)
    o_ref[0, :, W + 1 : W + 2, :] = jnp.zeros((H + 2, 1, half), dt)
    rows = TM // W
    for k in range(p_ref.shape[1] // TM):
        z = jnp.dot(
            p_ref[0, k * TM : (k + 1) * TM, :],
            w_ref[...],
            preferred_element_type=jnp.float32,
        )
        z = z + b_ref[...]
        o = (z[:, :half] * jax.nn.sigmoid(z[:, half:])).astype(o_ref.dtype)
        o3 = o.reshape(rows, W, half)
        r0 = 1 + k * rows
        o_ref[0, r0 : r0 + rows, 1 : W + 1, :] = o3


def _tap_slice(x_ref, im, oh0, TOH, Wo, C, tap, paired):
    if paired:
        di, pi, dj, q = tap
        sl = x_ref[im, oh0 + di : oh0 + di + TOH, pi,
                   dj : dj + Wo, q * C : (q + 1) * C]
    else:
        (i,) = tap
        sl = x_ref[im, oh0 + i : oh0 + i + TOH, :, :]
    return sl.reshape(TOH * Wo, C)


def _conv_pass1_kernel(x_ref, w_ref, y_ref, s1_ref, s2_ref, a_ref, *,
                       taps, Ho, Wo, C, TOH, IMGS, G, paired):
    M_img = Ho * Wo
    TMi = TOH * Wo
    R = G * TMi
    F2 = w_ref.shape[1]
    NT = min(256, F2)
    pieces = [(im, oh0) for im in range(IMGS) for oh0 in range(0, Ho, TOH)]
    n_chunks = len(pieces) // G
    nbuf = a_ref.shape[0]
    s1 = [jnp.zeros((1, NT), jnp.float32) for _ in range(F2 // NT)]
    s2 = [jnp.zeros((1, NT), jnp.float32) for _ in range(F2 // NT)]
    for g in range(n_chunks):
        buf = g % nbuf
        grp = pieces[g * G : (g + 1) * G]
        for gi, (im, oh0) in enumerate(grp):
            for t, tap in enumerate(taps):
                a_ref[buf, gi * TMi : (gi + 1) * TMi, t * C : (t + 1) * C] = (
                    _tap_slice(x_ref, im, oh0, TOH, Wo, C, tap, paired)
                )
        for nt in range(F2 // NT):
            z = jnp.dot(a_ref[buf], w_ref[:, nt * NT : (nt + 1) * NT],
                        preferred_element_type=jnp.float32)
            s1[nt] = s1[nt] + jnp.sum(z, axis=0, keepdims=True)
            s2[nt] = s2[nt] + jnp.sum(z * z, axis=0, keepdims=True)
            zb = z.astype(y_ref.dtype)
            for gi, (im, oh0) in enumerate(grp):
                y_ref[im, oh0 * Wo : oh0 * Wo + TMi, nt * NT : (nt + 1) * NT] = (
                    zb[gi * TMi : (gi + 1) * TMi, :]
                )
    for nt in range(F2 // NT):
        s1_ref[0, 0:1, nt * NT : (nt + 1) * NT] = s1[nt]
        s2_ref[0, 0:1, nt * NT : (nt + 1) * NT] = s2[nt]


_S2_TAPS = tuple((i // 2, i % 2, j // 2, j % 2) for i in range(3) for j in range(3))
_51_TAPS = tuple((i,) for i in range(5))


def _pass1_geometry(N, M_img, Ho, Wo):
    if M_img >= 512:
        IMGS, G, TOH = 1, 1, max(1, 512 // Wo)
    else:
        IMGS = max(1, min(N, 512 // M_img))
        G, TOH = IMGS, Ho
    return IMGS, G, TOH


def _bn_glu_pad_kernel(y_ref, s1_ref, s2_ref, g_ref, b_ref, o_ref, *, Ho, Wo,
                       TOH, ph, pw, M_tot):
    half = o_ref.shape[-1]
    Hp = o_ref.shape[1]
    Wp = o_ref.shape[2]
    dt = o_ref.dtype
    mean = jnp.sum(s1_ref[:, 0, :], axis=0, keepdims=True) / M_tot
    var = jnp.maximum(
        jnp.sum(s2_ref[:, 0, :], axis=0, keepdims=True) / M_tot - mean * mean, 0.0
    )
    sc = g_ref[...] * jax.lax.rsqrt(var + BN_EPS)
    sh = b_ref[...] - mean * sc
    if ph:
        o_ref[0, 0:ph, :, :] = jnp.zeros((ph, Wp, half), dt)
        o_ref[0, ph + Ho : Hp, :, :] = jnp.zeros((Hp - ph - Ho, Wp, half), dt)
    if pw:
        o_ref[0, :, 0:pw, :] = jnp.zeros((Hp, pw, half), dt)
        o_ref[0, :, pw + Wo : Wp, :] = jnp.zeros((Hp, Wp - pw - Wo, half), dt)
    TM = TOH * Wo
    for k in range(Ho // TOH):
        yv = y_ref[0, k * TM : (k + 1) * TM, :].astype(jnp.float32)
        yn = yv * sc + sh
        o = (yn[:, :half] * jax.nn.sigmoid(yn[:, half:])).astype(o_ref.dtype)
        o3 = o.reshape(TOH, Wo, half)
        r0 = ph + k * TOH
        o_ref[0, r0 : r0 + TOH, pw : pw + Wo, :] = o3


def _head_kernel(y_ref, s1_ref, s2_ref, g_ref, b_ref, c_ref, fw_ref, fb_ref,
                 ew_ref, eb_ref, o_ref, *, M_tot):
    N, M, F2 = y_ref.shape
    half = F2 // 2
    mean = jnp.sum(s1_ref[:, 0, :], axis=0, keepdims=True) / M_tot
    var = jnp.maximum(
        jnp.sum(s2_ref[:, 0, :], axis=0, keepdims=True) / M_tot - mean * mean, 0.0
    )
    sc = g_ref[...] * jax.lax.rsqrt(var + BN_EPS)
    sh = b_ref[...] - mean * sc
    chunk = min(128, M)
    ones = jnp.ones((1, chunk), jnp.float32)
    hs_rows = []
    for n in range(N):
        acc = jnp.zeros((1, half), jnp.float32)
        for k in range(M // chunk):
            yv = y_ref[n, k * chunk : (k + 1) * chunk, :].astype(jnp.float32)
            yn = yv * sc + sh
            h = yn[:, :half] * jax.nn.sigmoid(yn[:, half:])
            acc = acc + jnp.dot(ones, h, preferred_element_type=jnp.float32)
        hs_rows.append(acc)
    hs = jnp.concatenate(hs_rows, axis=0)
    e = jnp.dot(c_ref[...], ew_ref[...], preferred_element_type=jnp.float32) + eb_ref[...]
    t = hs * (e + fw_ref[...])
    o_ref[...] = jnp.sum(t, axis=1, keepdims=True) + fb_ref[...]


def _conv_s2_layer(xp, wmat, gamma, beta, Ho, Wo):
    N, Hp, Wp, C = xp.shape
    K, F2 = wmat.shape
    half = F2 // 2
    M_img = Ho * Wo
    IMGS, G, TOH = _pass1_geometry(N, M_img, Ho, Wo)
    R = G * TOH * Wo
    n_chunks = IMGS * M_img // R
    steps = N // IMGS
    x5 = xp.reshape(N, Hp // 2, 2, Wp // 2, 2 * C)

    y, s1p, s2p = pl.pallas_call(
        lambda x_ref, w_ref, y_ref, s1_ref, s2_ref, a_ref: _conv_pass1_kernel(
            x_ref, w_ref, y_ref, s1_ref, s2_ref, a_ref,
            taps=_S2_TAPS, Ho=Ho, Wo=Wo, C=C, TOH=TOH, IMGS=IMGS, G=G, paired=True,
        ),
        out_shape=(
            jax.ShapeDtypeStruct((N, M_img, F2), jnp.bfloat16),
            jax.ShapeDtypeStruct((steps, 1, F2), jnp.float32),
            jax.ShapeDtypeStruct((steps, 1, F2), jnp.float32),
        ),
        grid=(steps,),
        in_specs=[
            pl.BlockSpec((IMGS, Hp // 2, 2, Wp // 2, 2 * C),
                         lambda s: (s, 0, 0, 0, 0)),
            pl.BlockSpec((K, F2), lambda s: (0, 0)),
        ],
        out_specs=(
            pl.BlockSpec((IMGS, M_img, F2), lambda s: (s, 0, 0)),
            pl.BlockSpec((1, 1, F2), lambda s: (s, 0, 0)),
            pl.BlockSpec((1, 1, F2), lambda s: (s, 0, 0)),
        ),
        scratch_shapes=[pltpu.VMEM((2 if n_chunks > 1 else 1, R, K), jnp.bfloat16)],
        compiler_params=pltpu.CompilerParams(
            dimension_semantics=("parallel",), vmem_limit_bytes=_VMEM_LIMIT
        ),
        cost_estimate=pl.CostEstimate(
            flops=2 * N * M_img * K * F2,
            transcendentals=0,
            bytes_accessed=xp.size * 2 + wmat.size * 2 + N * M_img * F2 * 2,
        ),
    )(x5, wmat)
    return y, s1p, s2p, half


def _bn_glu_pad(y, s1p, s2p, gamma, beta, M_tot, Ho, Wo, half, ph, pw):
    N, M_img, F2 = y.shape
    steps = s1p.shape[0]
    TOH = min(4, Ho)
    Hp, Wp = Ho + 2 * ph, Wo + 2 * pw
    return pl.pallas_call(
        lambda y_ref, s1_ref, s2_ref, g_ref, b_ref, o_ref: _bn_glu_pad_kernel(
            y_ref, s1_ref, s2_ref, g_ref, b_ref, o_ref,
            Ho=Ho, Wo=Wo, TOH=TOH, ph=ph, pw=pw, M_tot=M_tot,
        ),
        out_shape=jax.ShapeDtypeStruct((N, Hp, Wp, half), jnp.bfloat16),
        grid=(N,),
        in_specs=[
            pl.BlockSpec((1, M_img, F2), lambda n: (n, 0, 0)),
            pl.BlockSpec((steps, 1, F2), lambda n: (0, 0, 0)),
            pl.BlockSpec((steps, 1, F2), lambda n: (0, 0, 0)),
            pl.BlockSpec((1, F2), lambda n: (0, 0)),
            pl.BlockSpec((1, F2), lambda n: (0, 0)),
        ],
        out_specs=pl.BlockSpec((1, Hp, Wp, half), lambda n: (n, 0, 0, 0)),
        compiler_params=pltpu.CompilerParams(
            dimension_semantics=("parallel",), vmem_limit_bytes=_VMEM_LIMIT
        ),
        cost_estimate=pl.CostEstimate(
            flops=3 * N * M_img * F2,
            transcendentals=N * M_img * half,
            bytes_accessed=y.size * 2 + N * Hp * Wp * half * 2,
        ),
    )(y, s1p, s2p, gamma, beta)


def kernel(x, c,
           layer0_wmat, layer0_bias,
           layer1_wmat, layer1_bias, layer1_gamma, layer1_beta,
           layer2_wmat, layer2_bias, layer2_gamma, layer2_beta,
           layer3_wmat, layer3_bias, layer3_gamma, layer3_beta,
           layer4_wmat, layer4_bias, layer4_gamma, layer4_beta,
           head_fout_w, head_fout_b, head_embed_w, head_embed_b):
    N, _, H, W = x.shape

    xb = jnp.pad(x.reshape(N, H, W).astype(jnp.bfloat16), ((0, 0), (1, 1), (1, 1)))
    taps = [xb[:, i : i + H, j : j + W] for i in range(3) for j in range(3)]
    patches = jnp.stack(taps, axis=-1).reshape(N, H * W, 9)
    F0 = layer0_wmat.shape[1]
    half0 = F0 // 2
    TM0 = min(1024, H * W)
    h0 = pl.pallas_call(
        lambda p_ref, w_ref, b_ref, o_ref: _l0_kernel(
            p_ref, w_ref, b_ref, o_ref, H=H, W=W, TM=TM0
        ),
        out_shape=jax.ShapeDtypeStruct((N, H + 2, W + 2, half0), jnp.bfloat16),
        grid=(N,),
        in_specs=[
            pl.BlockSpec((1, H * W, 9), lambda n: (n, 0, 0)),
            pl.BlockSpec((9, F0), lambda n: (0, 0)),
            pl.BlockSpec((1, F0), lambda n: (0, 0)),
        ],
        out_specs=pl.BlockSpec((1, H + 2, W + 2, half0), lambda n: (n, 0, 0, 0)),
        compiler_params=pltpu.CompilerParams(
            dimension_semantics=("parallel",), vmem_limit_bytes=_VMEM_LIMIT
        ),
        cost_estimate=pl.CostEstimate(
            flops=2 * N * H * W * 9 * F0,
            transcendentals=N * H * W * half0,
            bytes_accessed=patches.size * 2 + N * (H + 2) * (W + 2) * half0 * 2,
        ),
    )(patches, layer0_wmat, layer0_bias)

    xp = h0
    Ho, Wo = H, W
    for wmat, gamma, beta in (
        (layer1_wmat, layer1_gamma, layer1_beta),
        (layer2_wmat, layer2_gamma, layer2_beta),
        (layer3_wmat, layer3_gamma, layer3_beta),
    ):
        Ho, Wo = Ho // 2, Wo // 2
        y, s1p, s2p, half = _conv_s2_layer(xp, wmat, gamma, beta, Ho, Wo)
        last = wmat is layer3_wmat
        ph, pw = (2, 0) if last else (1, 1)
        xp = _bn_glu_pad(y, s1p, s2p, gamma, beta, N * Ho * Wo, Ho, Wo, half, ph, pw)

    N4, Hp4, Wp4, C4 = xp.shape
    K4, F4 = layer4_wmat.shape
    M4 = Ho * Wo
    IMGS4, G4, TOH4 = _pass1_geometry(N, M4, Ho, Wo)
    R4 = G4 * TOH4 * Wo
    n_chunks4 = IMGS4 * M4 // R4
    steps4 = N // IMGS4
    y4, s1p, s2p = pl.pallas_call(
        lambda x_ref, w_ref, y_ref, s1_ref, s2_ref, a_ref: _conv_pass1_kernel(
            x_ref, w_ref, y_ref, s1_ref, s2_ref, a_ref,
            taps=_51_TAPS, Ho=Ho, Wo=Wo, C=C4, TOH=TOH4, IMGS=IMGS4, G=G4,
            paired=False,
        ),
        out_shape=(
            jax.ShapeDtypeStruct((N, M4, F4), jnp.bfloat16),
            jax.ShapeDtypeStruct((steps4, 1, F4), jnp.float32),
            jax.ShapeDtypeStruct((steps4, 1, F4), jnp.float32),
        ),
        grid=(steps4,),
        in_specs=[
            pl.BlockSpec((IMGS4, Hp4, Wp4, C4), lambda s: (s, 0, 0, 0)),
            pl.BlockSpec((K4, F4), lambda s: (0, 0)),
        ],
        out_specs=(
            pl.BlockSpec((IMGS4, M4, F4), lambda s: (s, 0, 0)),
            pl.BlockSpec((1, 1, F4), lambda s: (s, 0, 0)),
            pl.BlockSpec((1, 1, F4), lambda s: (s, 0, 0)),
        ),
        scratch_shapes=[
            pltpu.VMEM((2 if n_chunks4 > 1 else 1, R4, K4), jnp.bfloat16)
        ],
        compiler_params=pltpu.CompilerParams(
            dimension_semantics=("parallel",), vmem_limit_bytes=_VMEM_LIMIT
        ),
        cost_estimate=pl.CostEstimate(
            flops=2 * N * M4 * K4 * F4,
            transcendentals=0,
            bytes_accessed=xp.size * 2 + layer4_wmat.size * 2 + N * M4 * F4 * 2,
        ),
    )(xp, layer4_wmat)

    fwT = head_fout_w.reshape(1, -1)
    out = pl.pallas_call(
        lambda *refs: _head_kernel(*refs, M_tot=N * M4),
        out_shape=jax.ShapeDtypeStruct((N, 1), jnp.float32),
        in_specs=[pl.BlockSpec(memory_space=pltpu.MemorySpace.VMEM)] * 10,
        out_specs=pl.BlockSpec(memory_space=pltpu.MemorySpace.VMEM),
        compiler_params=pltpu.CompilerParams(vmem_limit_bytes=_VMEM_LIMIT),
        cost_estimate=pl.CostEstimate(
            flops=6 * N * M4 * F4,
            transcendentals=N * M4 * F4 // 2,
            bytes_accessed=y4.size * 2,
        ),
    )(y4, s1p, s2p, layer4_gamma, layer4_beta, c.astype(jnp.float32), fwT,
      head_fout_b, head_embed_w, head_embed_b)
    return out


# speedup vs baseline: 13.1089x; 1.0189x over previous
import jax
import jax.numpy as jnp
from jax.experimental import pallas as pl
from jax.experimental.pallas import tpu as pltpu

BN_EPS = 1e-5
_VMEM_LIMIT = 48 * 1024 * 1024


def _l0_kernel(p_ref, w_ref, b_ref, o_ref, *, H, W, TM):
    half = o_ref.shape[-1]
    dt = o_ref.dtype
    Wp = W + 10
    o_ref[0, 0:1, :, :] = jnp.zeros((1, Wp, half), dt)
    o_ref[0, H + 1 : H + 2, :, :] = jnp.zeros((1, Wp, half), dt)
    o_ref[0, :, 0:8, :] = jnp.zeros((H + 2, 8, half), dt)
    o_ref[0, :, W + 8 : Wp, :] = jnp.zeros((H + 2, 2, half), dt)
    rows = TM // W
    for k in range(p_ref.shape[1] // TM):
        z = jnp.dot(
            p_ref[0, k * TM : (k + 1) * TM, :],
            w_ref[...],
            preferred_element_type=jnp.float32,
        )
        z = z + b_ref[...]
        o = (z[:, :half] * jax.nn.sigmoid(z[:, half:])).astype(o_ref.dtype)
        o3 = o.reshape(rows, W, half)
        r0 = 1 + k * rows
        o_ref[0, r0 : r0 + rows, 8 : W + 8, :] = o3


def _tap_slice(x_ref, im, oh0, TOH, Wo, C, tap, paired):
    if paired:
        di, pi, dj, q = tap
        sl = x_ref[im, oh0 + di : oh0 + di + TOH, pi,
                   dj : dj + Wo, q * C : (q + 1) * C]
    else:
        (i,) = tap
        sl = x_ref[im, oh0 + i : oh0 + i + TOH, :, :]
    return sl.reshape(TOH * Wo, C)


def _conv_pass1_kernel(x_ref, w_ref, y_ref, s1_ref, s2_ref, a_ref, *,
                       taps, Ho, Wo, C, TOH, IMGS, G, paired):
    M_img = Ho * Wo
    TMi = TOH * Wo
    R = G * TMi
    F2 = w_ref.shape[1]
    NT = min(256, F2)
    pieces = [(im, oh0) for im in range(IMGS) for oh0 in range(0, Ho, TOH)]
    n_chunks = len(pieces) // G
    nbuf = a_ref.shape[0]
    s1 = [jnp.zeros((1, NT), jnp.float32) for _ in range(F2 // NT)]
    s2 = [jnp.zeros((1, NT), jnp.float32) for _ in range(F2 // NT)]
    for g in range(n_chunks):
        buf = g % nbuf
        grp = pieces[g * G : (g + 1) * G]
        for gi, (im, oh0) in enumerate(grp):
            for t, tap in enumerate(taps):
                a_ref[buf, gi * TMi : (gi + 1) * TMi, t * C : (t + 1) * C] = (
                    _tap_slice(x_ref, im, oh0, TOH, Wo, C, tap, paired)
                )
        for nt in range(F2 // NT):
            z = jnp.dot(a_ref[buf], w_ref[:, nt * NT : (nt + 1) * NT],
                        preferred_element_type=jnp.float32)
            s1[nt] = s1[nt] + jnp.sum(z, axis=0, keepdims=True)
            s2[nt] = s2[nt] + jnp.sum(z * z, axis=0, keepdims=True)
            zb = z.astype(y_ref.dtype)
            for gi, (im, oh0) in enumerate(grp):
                y_ref[im, oh0 * Wo : oh0 * Wo + TMi, nt * NT : (nt + 1) * NT] = (
                    zb[gi * TMi : (gi + 1) * TMi, :]
                )
    for nt in range(F2 // NT):
        s1_ref[0, 0:1, nt * NT : (nt + 1) * NT] = s1[nt]
        s2_ref[0, 0:1, nt * NT : (nt + 1) * NT] = s2[nt]


_S2_TAPS = tuple((i // 2, i % 2, (j + 7) // 2, (j + 7) % 2)
                 for i in range(3) for j in range(3))
_51_TAPS = tuple((i,) for i in range(5))


def _pass1_geometry(N, M_img, Ho, Wo):
    if M_img >= 512:
        IMGS, G, TOH = 1, 1, max(1, 512 // Wo)
    else:
        IMGS = max(1, min(N, 512 // M_img))
        G, TOH = IMGS, Ho
    return IMGS, G, TOH


def _bn_glu_pad_kernel(y_ref, s1_ref, s2_ref, g_ref, b_ref, o_ref, *, Ho, Wo,
                       TOH, ph, pwl, pwr, M_tot):
    half = o_ref.shape[-1]
    Hp = o_ref.shape[1]
    Wp = o_ref.shape[2]
    dt = o_ref.dtype
    mean = jnp.sum(s1_ref[:, 0, :], axis=0, keepdims=True) / M_tot
    var = jnp.maximum(
        jnp.sum(s2_ref[:, 0, :], axis=0, keepdims=True) / M_tot - mean * mean, 0.0
    )
    sc = g_ref[...] * jax.lax.rsqrt(var + BN_EPS)
    sh = b_ref[...] - mean * sc
    if ph:
        o_ref[0, 0:ph, :, :] = jnp.zeros((ph, Wp, half), dt)
        o_ref[0, ph + Ho : Hp, :, :] = jnp.zeros((Hp - ph - Ho, Wp, half), dt)
    if pwl:
        o_ref[0, :, 0:pwl, :] = jnp.zeros((Hp, pwl, half), dt)
    if pwr:
        o_ref[0, :, pwl + Wo : Wp, :] = jnp.zeros((Hp, pwr, half), dt)
    TM = TOH * Wo
    for k in range(Ho // TOH):
        yv = y_ref[0, k * TM : (k + 1) * TM, :].astype(jnp.float32)
        yn = yv * sc + sh
        o = (yn[:, :half] * jax.nn.sigmoid(yn[:, half:])).astype(o_ref.dtype)
        o3 = o.reshape(TOH, Wo, half)
        r0 = ph + k * TOH
        o_ref[0, r0 : r0 + TOH, pwl : pwl + Wo, :] = o3


def _head_kernel(y_ref, s1_ref, s2_ref, g_ref, b_ref, c_ref, fw_ref, fb_ref,
                 ew_ref, eb_ref, o_ref, *, M_tot):
    N, M, F2 = y_ref.shape
    half = F2 // 2
    mean = jnp.sum(s1_ref[:, 0, :], axis=0, keepdims=True) / M_tot
    var = jnp.maximum(
        jnp.sum(s2_ref[:, 0, :], axis=0, keepdims=True) / M_tot - mean * mean, 0.0
    )
    sc = g_ref[...] * jax.lax.rsqrt(var + BN_EPS)
    sh = b_ref[...] - mean * sc
    chunk = min(128, M)
    ones = jnp.ones((1, chunk), jnp.float32)
    hs_rows = []
    for n in range(N):
        acc = jnp.zeros((1, half), jnp.float32)
        for k in range(M // chunk):
            yv = y_ref[n, k * chunk : (k + 1) * chunk, :].astype(jnp.float32)
            yn = yv * sc + sh
            h = yn[:, :half] * jax.nn.sigmoid(yn[:, half:])
            acc = acc + jnp.dot(ones, h, preferred_element_type=jnp.float32)
        hs_rows.append(acc)
    hs = jnp.concatenate(hs_rows, axis=0)
    e = jnp.dot(c_ref[...], ew_ref[...], preferred_element_type=jnp.float32) + eb_ref[...]
    t = hs * (e + fw_ref[...])
    o_ref[...] = jnp.sum(t, axis=1, keepdims=True) + fb_ref[...]


def _conv_s2_layer(xp, wmat, gamma, beta, Ho, Wo):
    N, Hp, Wp, C = xp.shape
    K, F2 = wmat.shape
    half = F2 // 2
    M_img = Ho * Wo
    IMGS, G, TOH = _pass1_geometry(N, M_img, Ho, Wo)
    R = G * TOH * Wo
    n_chunks = IMGS * M_img // R
    steps = N // IMGS
    x5 = xp.reshape(N, Hp // 2, 2, Wp // 2, 2 * C)

    y, s1p, s2p = pl.pallas_call(
        lambda x_ref, w_ref, y_ref, s1_ref, s2_ref, a_ref: _conv_pass1_kernel(
            x_ref, w_ref, y_ref, s1_ref, s2_ref, a_ref,
            taps=_S2_TAPS, Ho=Ho, Wo=Wo, C=C, TOH=TOH, IMGS=IMGS, G=G, paired=True,
        ),
        out_shape=(
            jax.ShapeDtypeStruct((N, M_img, F2), jnp.bfloat16),
            jax.ShapeDtypeStruct((steps, 1, F2), jnp.float32),
            jax.ShapeDtypeStruct((steps, 1, F2), jnp.float32),
        ),
        grid=(steps,),
        in_specs=[
            pl.BlockSpec((IMGS, Hp // 2, 2, Wp // 2, 2 * C),
                         lambda s: (s, 0, 0, 0, 0)),
            pl.BlockSpec((K, F2), lambda s: (0, 0)),
        ],
        out_specs=(
            pl.BlockSpec((IMGS, M_img, F2), lambda s: (s, 0, 0)),
            pl.BlockSpec((1, 1, F2), lambda s: (s, 0, 0)),
            pl.BlockSpec((1, 1, F2), lambda s: (s, 0, 0)),
        ),
        scratch_shapes=[pltpu.VMEM((2 if n_chunks > 1 else 1, R, K), jnp.bfloat16)],
        compiler_params=pltpu.CompilerParams(
            dimension_semantics=("parallel",), vmem_limit_bytes=_VMEM_LIMIT
        ),
        cost_estimate=pl.CostEstimate(
            flops=2 * N * M_img * K * F2,
            transcendentals=0,
            bytes_accessed=xp.size * 2 + wmat.size * 2 + N * M_img * F2 * 2,
        ),
    )(x5, wmat)
    return y, s1p, s2p, half


def _bn_glu_pad(y, s1p, s2p, gamma, beta, M_tot, Ho, Wo, half, ph, pwl, pwr):
    N, M_img, F2 = y.shape
    steps = s1p.shape[0]
    TOH = min(4, Ho)
    Hp, Wp = Ho + 2 * ph, pwl + Wo + pwr
    return pl.pallas_call(
        lambda y_ref, s1_ref, s2_ref, g_ref, b_ref, o_ref: _bn_glu_pad_kernel(
            y_ref, s1_ref, s2_ref, g_ref, b_ref, o_ref,
            Ho=Ho, Wo=Wo, TOH=TOH, ph=ph, pwl=pwl, pwr=pwr, M_tot=M_tot,
        ),
        out_shape=jax.ShapeDtypeStruct((N, Hp, Wp, half), jnp.bfloat16),
        grid=(N,),
        in_specs=[
            pl.BlockSpec((1, M_img, F2), lambda n: (n, 0, 0)),
            pl.BlockSpec((steps, 1, F2), lambda n: (0, 0, 0)),
            pl.BlockSpec((steps, 1, F2), lambda n: (0, 0, 0)),
            pl.BlockSpec((1, F2), lambda n: (0, 0)),
            pl.BlockSpec((1, F2), lambda n: (0, 0)),
        ],
        out_specs=pl.BlockSpec((1, Hp, Wp, half), lambda n: (n, 0, 0, 0)),
        compiler_params=pltpu.CompilerParams(
            dimension_semantics=("parallel",), vmem_limit_bytes=_VMEM_LIMIT
        ),
        cost_estimate=pl.CostEstimate(
            flops=3 * N * M_img * F2,
            transcendentals=N * M_img * half,
            bytes_accessed=y.size * 2 + N * Hp * Wp * half * 2,
        ),
    )(y, s1p, s2p, gamma, beta)


def kernel(x, c,
           layer0_wmat, layer0_bias,
           layer1_wmat, layer1_bias, layer1_gamma, layer1_beta,
           layer2_wmat, layer2_bias, layer2_gamma, layer2_beta,
           layer3_wmat, layer3_bias, layer3_gamma, layer3_beta,
           layer4_wmat, layer4_bias, layer4_gamma, layer4_beta,
           head_fout_w, head_fout_b, head_embed_w, head_embed_b):
    N, _, H, W = x.shape

    xb = jnp.pad(x.reshape(N, H, W).astype(jnp.bfloat16), ((0, 0), (1, 1), (1, 1)))
    taps = [xb[:, i : i + H, j : j + W] for i in range(3) for j in range(3)]
    patches = jnp.stack(taps, axis=-1).reshape(N, H * W, 9)
    F0 = layer0_wmat.shape[1]
    half0 = F0 // 2
    TM0 = min(1024, H * W)
    h0 = pl.pallas_call(
        lambda p_ref, w_ref, b_ref, o_ref: _l0_kernel(
            p_ref, w_ref, b_ref, o_ref, H=H, W=W, TM=TM0
        ),
        out_shape=jax.ShapeDtypeStruct((N, H + 2, W + 10, half0), jnp.bfloat16),
        grid=(N,),
        in_specs=[
            pl.BlockSpec((1, H * W, 9), lambda n: (n, 0, 0)),
            pl.BlockSpec((9, F0), lambda n: (0, 0)),
            pl.BlockSpec((1, F0), lambda n: (0, 0)),
        ],
        out_specs=pl.BlockSpec((1, H + 2, W + 10, half0), lambda n: (n, 0, 0, 0)),
        compiler_params=pltpu.CompilerParams(
            dimension_semantics=("parallel",), vmem_limit_bytes=_VMEM_LIMIT
        ),
        cost_estimate=pl.CostEstimate(
            flops=2 * N * H * W * 9 * F0,
            transcendentals=N * H * W * half0,
            bytes_accessed=patches.size * 2 + N * (H + 2) * (W + 10) * half0 * 2,
        ),
    )(patches, layer0_wmat, layer0_bias)

    xp = h0
    Ho, Wo = H, W
    for wmat, gamma, beta in (
        (layer1_wmat, layer1_gamma, layer1_beta),
        (layer2_wmat, layer2_gamma, layer2_beta),
        (layer3_wmat, layer3_gamma, layer3_beta),
    ):
        Ho, Wo = Ho // 2, Wo // 2
        y, s1p, s2p, half = _conv_s2_layer(xp, wmat, gamma, beta, Ho, Wo)
        last = wmat is layer3_wmat
        ph, pwl, pwr = (2, 0, 0) if last else (1, 8, 2)
        xp = _bn_glu_pad(y, s1p, s2p, gamma, beta, N * Ho * Wo, Ho, Wo, half,
                         ph, pwl, pwr)

    N4, Hp4, Wp4, C4 = xp.shape
    K4, F4 = layer4_wmat.shape
    M4 = Ho * Wo
    IMGS4, G4, TOH4 = _pass1_geometry(N, M4, Ho, Wo)
    R4 = G4 * TOH4 * Wo
    n_chunks4 = IMGS4 * M4 // R4
    steps4 = N // IMGS4
    y4, s1p, s2p = pl.pallas_call(
        lambda x_ref, w_ref, y_ref, s1_ref, s2_ref, a_ref: _conv_pass1_kernel(
            x_ref, w_ref, y_ref, s1_ref, s2_ref, a_ref,
            taps=_51_TAPS, Ho=Ho, Wo=Wo, C=C4, TOH=TOH4, IMGS=IMGS4, G=G4,
            paired=False,
        ),
        out_shape=(
            jax.ShapeDtypeStruct((N, M4, F4), jnp.bfloat16),
            jax.ShapeDtypeStruct((steps4, 1, F4), jnp.float32),
            jax.ShapeDtypeStruct((steps4, 1, F4), jnp.float32),
        ),
        grid=(steps4,),
        in_specs=[
            pl.BlockSpec((IMGS4, Hp4, Wp4, C4), lambda s: (s, 0, 0, 0)),
            pl.BlockSpec((K4, F4), lambda s: (0, 0)),
        ],
        out_specs=(
            pl.BlockSpec((IMGS4, M4, F4), lambda s: (s, 0, 0)),
            pl.BlockSpec((1, 1, F4), lambda s: (s, 0, 0)),
            pl.BlockSpec((1, 1, F4), lambda s: (s, 0, 0)),
        ),
        scratch_shapes=[
            pltpu.VMEM((2 if n_chunks4 > 1 else 1, R4, K4), jnp.bfloat16)
        ],
        compiler_params=pltpu.CompilerParams(
            dimension_semantics=("parallel",), vmem_limit_bytes=_VMEM_LIMIT
        ),
        cost_estimate=pl.CostEstimate(
            flops=2 * N * M4 * K4 * F4,
            transcendentals=0,
            bytes_accessed=xp.size * 2 + layer4_wmat.size * 2 + N * M4 * F4 * 2,
        ),
    )(xp, layer4_wmat)

    fwT = head_fout_w.reshape(1, -1)
    out = pl.pallas_call(
        lambda *refs: _head_kernel(*refs, M_tot=N * M4),
        out_shape=jax.ShapeDtypeStruct((N, 1), jnp.float32),
        in_specs=[pl.BlockSpec(memory_space=pltpu.MemorySpace.VMEM)] * 10,
        out_specs=pl.BlockSpec(memory_space=pltpu.MemorySpace.VMEM),
        compiler_params=pltpu.CompilerParams(vmem_limit_bytes=_VMEM_LIMIT),
        cost_estimate=pl.CostEstimate(
            flops=6 * N * M4 * F4,
            transcendentals=N * M4 * F4 // 2,
            bytes_accessed=y4.size * 2,
        ),
    )(y4, s1p, s2p, layer4_gamma, layer4_beta, c.astype(jnp.float32), fwT,
      head_fout_b, head_embed_w, head_embed_b)
    return out
